# baseline (speedup 1.0000x reference)
import jax
import jax.numpy as jnp
from jax.experimental import pallas as pl
from jax.experimental.pallas import tpu as pltpu

_CTXW = 20
_D = 128
_H = 128
_V = 100000
_R = 4096
_T = (_V + _R - 1) // _R


def _l1_kernel(idx_ref, tab_ref, w1_ref, b1_ref, h_ref, acc_ref):
    i = pl.program_id(0)
    part = jnp.dot(tab_ref[0], w1_ref[...].T,
                   preferred_element_type=jnp.float32)

    @pl.when(i == 0)
    def _init():
        acc_ref[...] = part

    @pl.when(i > 0)
    def _acc():
        acc_ref[...] += part

    @pl.when(i == _CTXW - 1)
    def _fin():
        h_ref[...] = jnp.maximum(acc_ref[...] + b1_ref[...], 0.0)


def _l2_kernel(h_ref, w2_ref, b2_ref, logits_ref, lse_ref, m_ref, s_ref):
    t = pl.program_id(0)
    logits = jnp.dot(h_ref[...], w2_ref[...].T,
                     preferred_element_type=jnp.float32) + b2_ref[...]
    col = t * _R + jax.lax.broadcasted_iota(jnp.int32, (1, _R), 1)
    logits = jnp.where(col < _V, logits, -jnp.inf)
    logits_ref[...] = logits

    tile_max = jnp.max(logits)

    @pl.when(t == 0)
    def _init():
        m_ref[0, 0] = tile_max
        s_ref[0, 0] = jnp.sum(jnp.exp(logits - tile_max))

    @pl.when(t > 0)
    def _acc():
        m_old = m_ref[0, 0]
        m_new = jnp.maximum(m_old, tile_max)
        s_ref[0, 0] = (s_ref[0, 0] * jnp.exp(m_old - m_new)
                       + jnp.sum(jnp.exp(logits - m_new)))
        m_ref[0, 0] = m_new

    @pl.when(t == _T - 1)
    def _fin():
        lse_ref[0, 0] = m_ref[0, 0] + jnp.log(s_ref[0, 0])


def _sub_kernel(lse_ref, logits_ref, out_ref):
    out_ref[...] = logits_ref[...] - lse_ref[0, 0]


def kernel(inputs, table, W1, b1, W2, b2):
    idx = inputs.astype(jnp.int32)
    tab3 = table.reshape(_V, 1, _D)
    b1r = b1.reshape(1, _H)
    b2r = b2.reshape(1, _V)

    h = pl.pallas_call(
        _l1_kernel,
        grid_spec=pltpu.PrefetchScalarGridSpec(
            num_scalar_prefetch=1,
            grid=(_CTXW,),
            in_specs=[
                pl.BlockSpec((1, 1, _D), lambda i, idx_ref: (idx_ref[i], 0, 0)),
                pl.BlockSpec((_H, _D), lambda i, idx_ref: (0, i)),
                pl.BlockSpec((1, _H), lambda i, idx_ref: (0, 0)),
            ],
            out_specs=pl.BlockSpec((1, _H), lambda i, idx_ref: (0, 0)),
            scratch_shapes=[pltpu.VMEM((1, _H), jnp.float32)],
        ),
        out_shape=jax.ShapeDtypeStruct((1, _H), jnp.float32),
    )(idx, tab3, W1, b1r)

    logits, lse = pl.pallas_call(
        _l2_kernel,
        grid=(_T,),
        in_specs=[
            pl.BlockSpec((1, _H), lambda t: (0, 0)),
            pl.BlockSpec((_R, _D), lambda t: (t, 0)),
            pl.BlockSpec((1, _R), lambda t: (0, t)),
        ],
        out_specs=[
            pl.BlockSpec((1, _R), lambda t: (0, t)),
            pl.BlockSpec(memory_space=pltpu.SMEM),
        ],
        out_shape=[
            jax.ShapeDtypeStruct((1, _V), jnp.float32),
            jax.ShapeDtypeStruct((1, 1), jnp.float32),
        ],
        scratch_shapes=[
            pltpu.SMEM((1, 1), jnp.float32),
            pltpu.SMEM((1, 1), jnp.float32),
        ],
    )(h, W2, b2r)

    out = pl.pallas_call(
        _sub_kernel,
        grid=(_T,),
        in_specs=[
            pl.BlockSpec(memory_space=pltpu.SMEM),
            pl.BlockSpec((1, _R), lambda t: (0, t)),
        ],
        out_specs=pl.BlockSpec((1, _R), lambda t: (0, t)),
        out_shape=jax.ShapeDtypeStruct((1, _V), jnp.float32),
    )(lse, logits)

    return out


# speedup vs baseline: 1.3453x; 1.3453x over previous
import jax
import jax.numpy as jnp
from jax.experimental import pallas as pl
from jax.experimental.pallas import tpu as pltpu

_CTXW = 20
_D = 128
_H = 128
_V = 100000
_R = 4096
_T = (_V + _R - 1) // _R


def _l1_kernel(idx_ref, tab_ref, w1_ref, b1_ref, h_ref, e_ref, sems):
    for j in range(_CTXW):
        pltpu.make_async_copy(
            tab_ref.at[pl.ds(idx_ref[j], 1), :],
            e_ref.at[:, pl.ds(j * _D, _D)],
            sems.at[j],
        ).start()
    for j in range(_CTXW):
        pltpu.make_async_copy(
            tab_ref.at[pl.ds(idx_ref[j], 1), :],
            e_ref.at[:, pl.ds(j * _D, _D)],
            sems.at[j],
        ).wait()
    h = jnp.dot(e_ref[...], w1_ref[...].T, preferred_element_type=jnp.float32)
    h_ref[...] = jnp.maximum(h + b1_ref[...], 0.0)


def _l2_kernel(h_ref, w2_ref, b2_ref, out_ref, logits_ref, m_ref, s_ref):
    t = pl.program_id(0)

    @pl.when(t < _T)
    def _stream():
        logits = jnp.dot(h_ref[...], w2_ref[...].T,
                         preferred_element_type=jnp.float32) + b2_ref[...]
        col = t * _R + jax.lax.broadcasted_iota(jnp.int32, (1, _R), 1)
        logits = jnp.where(col < _V, logits, -jnp.inf)
        logits_ref[:, pl.ds(t * _R, _R)] = logits

        tile_max = jnp.max(logits)

        @pl.when(t == 0)
        def _init():
            m_ref[0, 0] = tile_max
            s_ref[0, 0] = jnp.sum(jnp.exp(logits - tile_max))

        @pl.when(t > 0)
        def _acc():
            m_old = m_ref[0, 0]
            m_new = jnp.maximum(m_old, tile_max)
            s_ref[0, 0] = (s_ref[0, 0] * jnp.exp(m_old - m_new)
                           + jnp.sum(jnp.exp(logits - m_new)))
            m_ref[0, 0] = m_new

        @pl.when(t == _T - 1)
        def _fin():
            m_ref[0, 0] = m_ref[0, 0] + jnp.log(s_ref[0, 0])

    @pl.when(t >= _T)
    def _emit():
        tt = t - _T
        out_ref[...] = logits_ref[:, pl.ds(tt * _R, _R)] - m_ref[0, 0]


def kernel(inputs, table, W1, b1, W2, b2):
    idx = inputs.astype(jnp.int32)
    b1r = b1.reshape(1, _H)
    b2r = b2.reshape(1, _V)

    h = pl.pallas_call(
        _l1_kernel,
        in_specs=[
            pl.BlockSpec(memory_space=pltpu.SMEM),
            pl.BlockSpec(memory_space=pl.ANY),
            pl.BlockSpec(memory_space=pltpu.VMEM),
            pl.BlockSpec(memory_space=pltpu.VMEM),
        ],
        out_specs=pl.BlockSpec(memory_space=pltpu.VMEM),
        out_shape=jax.ShapeDtypeStruct((1, _H), jnp.float32),
        scratch_shapes=[
            pltpu.VMEM((1, _CTXW * _D), jnp.float32),
            pltpu.SemaphoreType.DMA((_CTXW,)),
        ],
    )(idx, table, W1, b1r)

    out = pl.pallas_call(
        _l2_kernel,
        grid=(2 * _T,),
        in_specs=[
            pl.BlockSpec((1, _H), lambda t: (0, 0)),
            pl.BlockSpec((_R, _D), lambda t: (jnp.minimum(t, _T - 1), 0)),
            pl.BlockSpec((1, _R), lambda t: (0, jnp.minimum(t, _T - 1))),
        ],
        out_specs=pl.BlockSpec((1, _R), lambda t: (0, jnp.maximum(t - _T, 0))),
        out_shape=jax.ShapeDtypeStruct((1, _V), jnp.float32),
        scratch_shapes=[
            pltpu.VMEM((1, _T * _R), jnp.float32),
            pltpu.SMEM((1, 1), jnp.float32),
            pltpu.SMEM((1, 1), jnp.float32),
        ],
    )(h, W2, b2r)

    return out


# speedup vs baseline: 1.5710x; 1.1678x over previous
import jax
import jax.numpy as jnp
from jax.experimental import pallas as pl
from jax.experimental.pallas import tpu as pltpu

_CTXW = 20
_D = 128
_H = 128
_V = 100000
_R = 4096
_NB = (_V + _R - 1) // _R
_S = 2
_P1 = (_NB + _S - 1) // _S
_OFFS = [min(k * _P1, _NB) for k in range(_S + 1)]


def _l1_kernel(idx_ref, tab_ref, w1_ref, b1_ref, h_ref, e_ref, sems):
    for j in range(_CTXW):
        pltpu.make_async_copy(
            tab_ref.at[pl.ds(idx_ref[j], 1), :],
            e_ref.at[:, pl.ds(j * _D, _D)],
            sems.at[j],
        ).start()
    for j in range(_CTXW):
        pltpu.make_async_copy(
            tab_ref.at[pl.ds(idx_ref[j], 1), :],
            e_ref.at[:, pl.ds(j * _D, _D)],
            sems.at[j],
        ).wait()
    h = jnp.dot(e_ref[...], w1_ref[...].T, preferred_element_type=jnp.float32)
    h_ref[...] = jnp.maximum(h + b1_ref[...], 0.0)


def _l2_kernel(h_ref, *refs):
    w2_refs = refs[:_S]
    b2_refs = refs[_S:2 * _S]
    out_ref = refs[2 * _S]
    logits_ref, m_ref, s_ref = refs[2 * _S + 1:]
    t = pl.program_id(0)

    @pl.when(t == 0)
    def _init():
        m_ref[0, 0] = -jnp.inf
        s_ref[0, 0] = 0.0

    @pl.when(t < _P1)
    def _stream():
        h = h_ref[...]
        for k in range(_S):
            cnt = _OFFS[k + 1] - _OFFS[k]

            @pl.when(t < cnt)
            def _do(k=k):
                b = _OFFS[k] + t
                logits = jnp.dot(h, w2_refs[k][...].T,
                                 preferred_element_type=jnp.float32)
                logits = logits + b2_refs[k][...]
                col = b * _R + jax.lax.broadcasted_iota(jnp.int32, (1, _R), 1)
                logits = jnp.where(col < _V, logits, -jnp.inf)
                logits_ref[:, pl.ds(b * _R, _R)] = logits

                tile_max = jnp.max(logits)
                m_old = m_ref[0, 0]
                m_new = jnp.maximum(m_old, tile_max)
                s_ref[0, 0] = (s_ref[0, 0] * jnp.exp(m_old - m_new)
                               + jnp.sum(jnp.exp(logits - m_new)))
                m_ref[0, 0] = m_new

    @pl.when(t == _P1 - 1)
    def _fin():
        m_ref[0, 0] = m_ref[0, 0] + jnp.log(s_ref[0, 0])

    @pl.when(t >= _P1)
    def _emit():
        tt = t - _P1
        out_ref[...] = logits_ref[:, pl.ds(tt * _R, _R)] - m_ref[0, 0]


def kernel(inputs, table, W1, b1, W2, b2):
    idx = inputs.astype(jnp.int32)
    b1r = b1.reshape(1, _H)
    b2r = b2.reshape(1, _V)

    h = pl.pallas_call(
        _l1_kernel,
        in_specs=[
            pl.BlockSpec(memory_space=pltpu.SMEM),
            pl.BlockSpec(memory_space=pl.ANY),
            pl.BlockSpec(memory_space=pltpu.VMEM),
            pl.BlockSpec(memory_space=pltpu.VMEM),
        ],
        out_specs=pl.BlockSpec(memory_space=pltpu.VMEM),
        out_shape=jax.ShapeDtypeStruct((1, _H), jnp.float32),
        scratch_shapes=[
            pltpu.VMEM((1, _CTXW * _D), jnp.float32),
            pltpu.SemaphoreType.DMA((_CTXW,)),
        ],
    )(idx, table, W1, b1r)

    def _w2_map(k):
        lo, hi = _OFFS[k], _OFFS[k + 1] - 1
        return lambda t: (jnp.clip(lo + t, lo, hi), 0)

    def _b2_map(k):
        lo, hi = _OFFS[k], _OFFS[k + 1] - 1
        return lambda t: (0, jnp.clip(lo + t, lo, hi))

    out = pl.pallas_call(
        _l2_kernel,
        grid=(_P1 + _NB,),
        in_specs=(
            [pl.BlockSpec((1, _H), lambda t: (0, 0))]
            + [pl.BlockSpec((_R, _D), _w2_map(k)) for k in range(_S)]
            + [pl.BlockSpec((1, _R), _b2_map(k)) for k in range(_S)]
        ),
        out_specs=pl.BlockSpec(
            (1, _R), lambda t: (0, jnp.clip(t - _P1, 0, _NB - 1))),
        out_shape=jax.ShapeDtypeStruct((1, _V), jnp.float32),
        scratch_shapes=[
            pltpu.VMEM((1, _NB * _R), jnp.float32),
            pltpu.SMEM((1, 1), jnp.float32),
            pltpu.SMEM((1, 1), jnp.float32),
        ],
    )(h, *([W2] * _S), *([b2r] * _S))

    return out


# speedup vs baseline: 1.8048x; 1.1488x over previous
import jax
import jax.numpy as jnp
from jax.experimental import pallas as pl
from jax.experimental.pallas import tpu as pltpu

_CTXW = 20
_D = 128
_H = 128
_V = 100000
_R = 4096
_NB = (_V + _R - 1) // _R
_S = 4
_P1 = (_NB + _S - 1) // _S
_OFFS = [min(k * _P1, _NB) for k in range(_S + 1)]


def _l1_kernel(idx_ref, tab_ref, w1_ref, b1_ref, h_ref, e_ref, sems):
    for j in range(_CTXW):
        pltpu.make_async_copy(
            tab_ref.at[pl.ds(idx_ref[j], 1), :],
            e_ref.at[:, pl.ds(j * _D, _D)],
            sems.at[j],
        ).start()
    for j in range(_CTXW):
        pltpu.make_async_copy(
            tab_ref.at[pl.ds(idx_ref[j], 1), :],
            e_ref.at[:, pl.ds(j * _D, _D)],
            sems.at[j],
        ).wait()
    h = jnp.dot(e_ref[...], w1_ref[...].T, preferred_element_type=jnp.float32)
    h_ref[...] = jnp.maximum(h + b1_ref[...], 0.0)


def _l2_kernel(h_ref, *refs):
    w2_refs = refs[:_S]
    b2_refs = refs[_S:2 * _S]
    out_ref = refs[2 * _S]
    logits_ref, m_ref, s_ref = refs[2 * _S + 1:]
    t = pl.program_id(0)

    @pl.when(t == 0)
    def _init():
        m_ref[0, 0] = -jnp.inf
        s_ref[0, 0] = 0.0

    @pl.when(t < _P1)
    def _stream():
        h = h_ref[...]
        for k in range(_S):
            cnt = _OFFS[k + 1] - _OFFS[k]

            @pl.when(t < cnt)
            def _do(k=k):
                b = _OFFS[k] + t
                logits = jnp.dot(h, w2_refs[k][...].T,
                                 preferred_element_type=jnp.float32)
                logits = logits + b2_refs[k][...]
                col = b * _R + jax.lax.broadcasted_iota(jnp.int32, (1, _R), 1)
                logits = jnp.where(col < _V, logits, -jnp.inf)
                logits_ref[:, pl.ds(b * _R, _R)] = logits

                tile_max = jnp.max(logits)
                m_old = m_ref[0, 0]
                m_new = jnp.maximum(m_old, tile_max)
                s_ref[0, 0] = (s_ref[0, 0] * jnp.exp(m_old - m_new)
                               + jnp.sum(jnp.exp(logits - m_new)))
                m_ref[0, 0] = m_new

    @pl.when(t == _P1 - 1)
    def _fin():
        m_ref[0, 0] = m_ref[0, 0] + jnp.log(s_ref[0, 0])

    @pl.when(t >= _P1)
    def _emit():
        tt = t - _P1
        out_ref[...] = logits_ref[:, pl.ds(tt * _R, _R)] - m_ref[0, 0]


def kernel(inputs, table, W1, b1, W2, b2):
    idx = inputs.astype(jnp.int32)
    b1r = b1.reshape(1, _H)
    b2r = b2.reshape(1, _V)

    h = pl.pallas_call(
        _l1_kernel,
        in_specs=[
            pl.BlockSpec(memory_space=pltpu.SMEM),
            pl.BlockSpec(memory_space=pl.ANY),
            pl.BlockSpec(memory_space=pltpu.VMEM),
            pl.BlockSpec(memory_space=pltpu.VMEM),
        ],
        out_specs=pl.BlockSpec(memory_space=pltpu.VMEM),
        out_shape=jax.ShapeDtypeStruct((1, _H), jnp.float32),
        scratch_shapes=[
            pltpu.VMEM((1, _CTXW * _D), jnp.float32),
            pltpu.SemaphoreType.DMA((_CTXW,)),
        ],
    )(idx, table, W1, b1r)

    def _w2_map(k):
        lo, hi = _OFFS[k], _OFFS[k + 1] - 1
        return lambda t: (jnp.clip(lo + t, lo, hi), 0)

    def _b2_map(k):
        lo, hi = _OFFS[k], _OFFS[k + 1] - 1
        return lambda t: (0, jnp.clip(lo + t, lo, hi))

    out = pl.pallas_call(
        _l2_kernel,
        grid=(_P1 + _NB,),
        in_specs=(
            [pl.BlockSpec((1, _H), lambda t: (0, 0))]
            + [pl.BlockSpec((_R, _D), _w2_map(k)) for k in range(_S)]
            + [pl.BlockSpec((1, _R), _b2_map(k)) for k in range(_S)]
        ),
        out_specs=pl.BlockSpec(
            (1, _R), lambda t: (0, jnp.clip(t - _P1, 0, _NB - 1))),
        out_shape=jax.ShapeDtypeStruct((1, _V), jnp.float32),
        scratch_shapes=[
            pltpu.VMEM((1, _NB * _R), jnp.float32),
            pltpu.SMEM((1, 1), jnp.float32),
            pltpu.SMEM((1, 1), jnp.float32),
        ],
    )(h, *([W2] * _S), *([b2r] * _S))

    return out


# speedup vs baseline: 2.4628x; 1.3646x over previous
import jax
import jax.numpy as jnp
from jax.experimental import pallas as pl
from jax.experimental.pallas import tpu as pltpu

_CTXW = 20
_D = 128
_H = 128
_V = 100000
_R = 4096
_NB = (_V + _R - 1) // _R
_S = 5
_P1 = (_NB + _S - 1) // _S
_OFFS = [min(k * _P1, _NB) for k in range(_S + 1)]


def _l1_kernel(idx_ref, tab_ref, w1_ref, b1_ref, h_ref, e_ref, sems):
    for j in range(_CTXW):
        pltpu.make_async_copy(
            tab_ref.at[pl.ds(idx_ref[j], 1), :],
            e_ref.at[:, pl.ds(j * _D, _D)],
            sems.at[j],
        ).start()
    for j in range(_CTXW):
        pltpu.make_async_copy(
            tab_ref.at[pl.ds(idx_ref[j], 1), :],
            e_ref.at[:, pl.ds(j * _D, _D)],
            sems.at[j],
        ).wait()
    h = jnp.dot(e_ref[...], w1_ref[...].T, preferred_element_type=jnp.float32)
    h_ref[...] = jnp.maximum(h + b1_ref[...], 0.0)


def _l2_kernel(h_ref, *refs):
    w2_refs = refs[:_S]
    b2_refs = refs[_S:2 * _S]
    out_ref = refs[2 * _S]
    logits_ref, m_ref, s_ref = refs[2 * _S + 1:]
    t = pl.program_id(0)

    @pl.when(t == 0)
    def _init():
        m_ref[0, 0] = -jnp.inf
        s_ref[0, 0] = 0.0

    @pl.when(t < _P1)
    def _stream():
        h = h_ref[...]
        for k in range(_S):
            cnt = _OFFS[k + 1] - _OFFS[k]

            @pl.when(t < cnt)
            def _do(k=k):
                b = _OFFS[k] + t
                logits = jnp.dot(h, w2_refs[k][...].T,
                                 preferred_element_type=jnp.float32)
                logits = logits + b2_refs[k][...]
                col = b * _R + jax.lax.broadcasted_iota(jnp.int32, (1, _R), 1)
                logits = jnp.where(col < _V, logits, -jnp.inf)
                logits_ref[:, pl.ds(b * _R, _R)] = logits

                tile_max = jnp.max(logits)
                m_old = m_ref[0, 0]
                m_new = jnp.maximum(m_old, tile_max)
                s_ref[0, 0] = (s_ref[0, 0] * jnp.exp(m_old - m_new)
                               + jnp.sum(jnp.exp(logits - m_new)))
                m_ref[0, 0] = m_new

    @pl.when(t == _P1 - 1)
    def _fin():
        m_ref[0, 0] = m_ref[0, 0] + jnp.log(s_ref[0, 0])

    @pl.when(t == _P1)
    def _emit():
        out_ref[...] = logits_ref[:, :_V] - m_ref[0, 0]


def kernel(inputs, table, W1, b1, W2, b2):
    idx = inputs.astype(jnp.int32)
    b1r = b1.reshape(1, _H)
    b2r = b2.reshape(1, _V)

    h = pl.pallas_call(
        _l1_kernel,
        in_specs=[
            pl.BlockSpec(memory_space=pltpu.SMEM),
            pl.BlockSpec(memory_space=pl.ANY),
            pl.BlockSpec(memory_space=pltpu.VMEM),
            pl.BlockSpec(memory_space=pltpu.VMEM),
        ],
        out_specs=pl.BlockSpec(memory_space=pltpu.VMEM),
        out_shape=jax.ShapeDtypeStruct((1, _H), jnp.float32),
        scratch_shapes=[
            pltpu.VMEM((1, _CTXW * _D), jnp.float32),
            pltpu.SemaphoreType.DMA((_CTXW,)),
        ],
    )(idx, table, W1, b1r)

    def _w2_map(k):
        lo, hi = _OFFS[k], _OFFS[k + 1] - 1
        return lambda t: (jnp.clip(lo + t, lo, hi), 0)

    def _b2_map(k):
        lo, hi = _OFFS[k], _OFFS[k + 1] - 1
        return lambda t: (0, jnp.clip(lo + t, lo, hi))

    out = pl.pallas_call(
        _l2_kernel,
        grid=(_P1 + 1,),
        in_specs=(
            [pl.BlockSpec((1, _H), lambda t: (0, 0))]
            + [pl.BlockSpec((_R, _D), _w2_map(k)) for k in range(_S)]
            + [pl.BlockSpec((1, _R), _b2_map(k)) for k in range(_S)]
        ),
        out_specs=pl.BlockSpec((1, _V), lambda t: (0, 0)),
        out_shape=jax.ShapeDtypeStruct((1, _V), jnp.float32),
        scratch_shapes=[
            pltpu.VMEM((1, _NB * _R), jnp.float32),
            pltpu.SMEM((1, 1), jnp.float32),
            pltpu.SMEM((1, 1), jnp.float32),
        ],
    )(h, *([W2] * _S), *([b2r] * _S))

    return out
